# Initial kernel scaffold; baseline (speedup 1.0000x reference)
import functools

import jax
import jax.numpy as jnp
from jax import lax
from jax.experimental import pallas as pl
from jax.experimental.pallas import tpu as pltpu
from jax.experimental.pallas import tpu_sc as plsc

_INFO = plsc.get_sparse_core_info()
NC = _INFO.num_cores
NS = _INFO.num_subcores
L = _INFO.num_lanes
NW = NC * NS

N_DST = 10000
D = 128
BATCH = 128
EXTRA = 16
ACC_ROWS = N_DST + EXTRA
ROWS_PER_SUB = ACC_ROWS // NS


def _sc_partial_sums(src_emb, sidx, didx, nb):
    mesh = plsc.VectorSubcoreMesh(core_axis_name="c", subcore_axis_name="s")

    @functools.partial(
        pl.kernel,
        mesh=mesh,
        out_type=jax.ShapeDtypeStruct((NC, ACC_ROWS, D), jnp.float32),
        scratch_types=[
            pltpu.VMEM((nb, BATCH), jnp.int32),
            pltpu.VMEM((nb, BATCH), jnp.int32),
            pltpu.VMEM((BATCH, D), jnp.float32),
            pltpu.VMEM_SHARED((ACC_ROWS, D), jnp.float32),
            pltpu.SemaphoreType.DMA,
        ],
    )
    def body(src_hbm, sidx_hbm, didx_hbm, out_hbm, sidx_v, didx_v, rows_v,
             acc_sh, sem):
        cid = lax.axis_index("c")
        sid = lax.axis_index("s")
        wid = sid * NC + cid

        pltpu.sync_copy(sidx_hbm.at[wid], sidx_v)
        pltpu.sync_copy(didx_hbm.at[wid], didx_v)

        def zrow(i, carry):
            for c in range(D // L):
                rows_v[i, pl.ds(c * L, L)] = jnp.zeros((L,), jnp.float32)
            return carry

        lax.fori_loop(0, BATCH, zrow, 0)

        base = sid * ROWS_PER_SUB
        full = ROWS_PER_SUB // BATCH
        rem = ROWS_PER_SUB - full * BATCH
        for k in range(full):
            pltpu.sync_copy(rows_v, acc_sh.at[pl.ds(base + k * BATCH, BATCH)])
        if rem:
            pltpu.sync_copy(rows_v.at[pl.ds(0, rem)],
                            acc_sh.at[pl.ds(base + full * BATCH, rem)])
        plsc.subcore_barrier()

        def step(j, carry):
            pltpu.async_copy(src_hbm.at[sidx_v.at[j]], rows_v, sem).wait()
            pltpu.sync_copy(rows_v, acc_sh.at[didx_v.at[j]], add=True)
            return carry

        lax.fori_loop(0, nb, step, 0)
        plsc.subcore_barrier()

        pltpu.sync_copy(acc_sh.at[pl.ds(base, ROWS_PER_SUB)],
                        out_hbm.at[cid, pl.ds(base, ROWS_PER_SUB)])

    return body(src_emb, sidx, didx)


def _merge_partials(partials):
    blk = 400

    def body(p_ref, o_ref):
        o_ref[...] = jnp.sum(p_ref[...], axis=0)

    return pl.pallas_call(
        body,
        out_shape=jax.ShapeDtypeStruct((N_DST, D), jnp.float32),
        grid=(N_DST // blk,),
        in_specs=[pl.BlockSpec((NC, blk, D), lambda i: (0, i, 0))],
        out_specs=pl.BlockSpec((blk, D), lambda i: (i, 0)),
    )(partials)


def kernel(src_emb, edge_index):
    n_src = src_emb.shape[0]
    e = edge_index.shape[1]
    nb = -(-e // (NW * BATCH))
    e_pad = NW * BATCH * nb
    npad = e_pad - e

    ei = edge_index.astype(jnp.int32)
    pad = jnp.arange(npad, dtype=jnp.int32)
    sidx = jnp.concatenate([ei[0], pad % n_src]).reshape(NW, nb, BATCH)
    didx = jnp.concatenate([ei[1], N_DST + (pad % EXTRA)]).reshape(NW, nb, BATCH)

    partials = _sc_partial_sums(src_emb.astype(jnp.float32), sidx, didx, nb)
    return _merge_partials(partials)


# speedup vs baseline: 8.9691x; 8.9691x over previous
import functools

import jax
import jax.numpy as jnp
from jax import lax
from jax.experimental import pallas as pl
from jax.experimental.pallas import tpu as pltpu
from jax.experimental.pallas import tpu_sc as plsc

_INFO = plsc.get_sparse_core_info()
NC = _INFO.num_cores
NS = _INFO.num_subcores
L = _INFO.num_lanes
NW = NC * NS

N_DST = 10000
D = 128
BATCH = 128
EXTRA = 112
ACC_ROWS = N_DST + EXTRA
ROWS_PER_SUB = ACC_ROWS // NS


def _sc_partial_sums(src_emb, sidx, didx, nb):
    mesh = plsc.VectorSubcoreMesh(core_axis_name="c", subcore_axis_name="s")

    @functools.partial(
        pl.kernel,
        mesh=mesh,
        out_type=jax.ShapeDtypeStruct((NC, ACC_ROWS, D), jnp.float32),
        scratch_types=[
            pltpu.VMEM((nb, BATCH), jnp.int32),
            pltpu.VMEM((nb, BATCH), jnp.int32),
            pltpu.VMEM((BATCH, D), jnp.float32),
            pltpu.VMEM_SHARED((ACC_ROWS, D), jnp.float32),
            pltpu.SemaphoreType.DMA,
        ],
    )
    def body(src_hbm, sidx_hbm, didx_hbm, out_hbm, sidx_v, didx_v, rows_v,
             acc_sh, sem):
        cid = lax.axis_index("c")
        sid = lax.axis_index("s")
        wid = sid * NC + cid

        pltpu.sync_copy(sidx_hbm.at[wid], sidx_v)
        pltpu.sync_copy(didx_hbm.at[wid], didx_v)

        @pl.loop(jnp.int32(0), jnp.int32(BATCH))
        def _zrow(i):
            for c in range(D // L):
                rows_v[i, pl.ds(c * L, L)] = jnp.zeros((L,), jnp.float32)

        base = sid * ROWS_PER_SUB
        full = ROWS_PER_SUB // BATCH
        rem = ROWS_PER_SUB - full * BATCH
        for k in range(full):
            pltpu.sync_copy(rows_v, acc_sh.at[pl.ds(base + k * BATCH, BATCH)])
        if rem:
            pltpu.sync_copy(rows_v.at[pl.ds(0, rem)],
                            acc_sh.at[pl.ds(base + full * BATCH, rem)])
        plsc.subcore_barrier()

        @pl.loop(jnp.int32(0), jnp.int32(nb))
        def _step(j):
            pltpu.async_copy(src_hbm.at[sidx_v.at[j]], rows_v, sem).wait()
            pltpu.sync_copy(rows_v, acc_sh.at[didx_v.at[j]], add=True)
        plsc.subcore_barrier()

        pltpu.sync_copy(acc_sh.at[pl.ds(base, ROWS_PER_SUB)],
                        out_hbm.at[cid, pl.ds(base, ROWS_PER_SUB)])

    return body(src_emb, sidx, didx)


def _merge_partials(partials):
    blk = 400

    def body(p_ref, o_ref):
        o_ref[...] = jnp.sum(p_ref[...], axis=0)

    return pl.pallas_call(
        body,
        out_shape=jax.ShapeDtypeStruct((N_DST, D), jnp.float32),
        grid=(N_DST // blk,),
        in_specs=[pl.BlockSpec((NC, blk, D), lambda i: (i * 0, i, i * 0))],
        out_specs=pl.BlockSpec((blk, D), lambda i: (i, i * 0)),
    )(partials)


def kernel(src_emb, edge_index):
    n_src = src_emb.shape[0]
    e = edge_index.shape[1]
    nb = -(-e // (NW * BATCH))
    e_pad = NW * BATCH * nb
    npad = e_pad - e

    ei = edge_index.astype(jnp.int32)
    pad = jnp.arange(npad, dtype=jnp.int32)
    sidx = jnp.concatenate([ei[0], pad % n_src]).reshape(NW, nb, BATCH)
    didx = jnp.concatenate([ei[1], N_DST + (pad % EXTRA)]).reshape(NW, nb, BATCH)

    partials = _sc_partial_sums(src_emb.astype(jnp.float32), sidx, didx, nb)
    return _merge_partials(partials)


# speedup vs baseline: 13.0357x; 1.4534x over previous
import functools

import jax
import jax.numpy as jnp
from jax import lax
from jax.experimental import pallas as pl
from jax.experimental.pallas import tpu as pltpu
from jax.experimental.pallas import tpu_sc as plsc

_INFO = plsc.get_sparse_core_info()
NC = _INFO.num_cores
NS = _INFO.num_subcores
L = _INFO.num_lanes
NW = NC * NS

N_DST = 10000
D = 128
BATCH = 128
EXTRA = 112
ACC_ROWS = N_DST + EXTRA
ROWS_PER_SUB = ACC_ROWS // NS


def _sc_partial_sums(src_emb, eidx, nb):
    mesh = plsc.VectorSubcoreMesh(core_axis_name="c", subcore_axis_name="s")

    assert nb >= 3

    @functools.partial(
        pl.kernel,
        mesh=mesh,
        out_type=jax.ShapeDtypeStruct((NC, ACC_ROWS, D), jnp.float32),
        scratch_types=[
            pltpu.VMEM((4, 2, BATCH), jnp.int32),
            pltpu.VMEM((2, BATCH, D), jnp.float32),
            pltpu.VMEM_SHARED((ACC_ROWS, D), jnp.float32),
            pltpu.SemaphoreType.DMA,
            pltpu.SemaphoreType.DMA,
            pltpu.SemaphoreType.DMA,
            pltpu.SemaphoreType.DMA,
            pltpu.SemaphoreType.DMA,
            pltpu.SemaphoreType.DMA,
        ],
    )
    def body(src_hbm, eidx_hbm, out_hbm, ibufs, rows_v, acc_sh,
             is0, is1, is2, is3, gs0, gs1):
        cid = lax.axis_index("c")
        sid = lax.axis_index("s")
        wid = sid * NC + cid
        isems = (is0, is1, is2, is3)
        gsems = (gs0, gs1)

        def _ifetch(jb, k):
            return pltpu.make_async_copy(eidx_hbm.at[wid, jb],
                                         ibufs.at[jnp.int32(k)], isems[k])

        def _gath(jb, k, b):
            return pltpu.make_async_copy(
                src_hbm.at[ibufs.at[jnp.int32(k), jnp.int32(0)]],
                rows_v.at[jnp.int32(b)], gsems[b])

        @pl.loop(jnp.int32(0), jnp.int32(BATCH))
        def _zrow(i):
            for c in range(D // L):
                rows_v[jnp.int32(0), i, pl.ds(c * L, L)] = jnp.zeros(
                    (L,), jnp.float32)

        base = sid * ROWS_PER_SUB
        full = ROWS_PER_SUB // BATCH
        rem = ROWS_PER_SUB - full * BATCH
        for k in range(full):
            pltpu.sync_copy(rows_v.at[jnp.int32(0)],
                            acc_sh.at[pl.ds(base + k * BATCH, BATCH)])
        if rem:
            pltpu.sync_copy(rows_v.at[jnp.int32(0)].at[pl.ds(0, rem)],
                            acc_sh.at[pl.ds(base + full * BATCH, rem)])
        plsc.subcore_barrier()

        _ifetch(jnp.int32(0), 0).start()
        _ifetch(jnp.int32(1), 1).start()
        _ifetch(jnp.int32(2), 2).start()
        _ifetch(jnp.int32(0), 0).wait()
        _gath(jnp.int32(0), 0, 0).start()
        _ifetch(jnp.int32(1), 1).wait()
        _gath(jnp.int32(1), 1, 1).start()

        @pl.loop(jnp.int32(0), jnp.int32(nb), step=jnp.int32(4))
        def _step(j):
            for b in range(4):
                jb = j + b

                def _one(jb=jb, b=b):
                    _gath(jb, b, b & 1).wait()
                    pltpu.sync_copy(
                        rows_v.at[jnp.int32(b & 1)],
                        acc_sh.at[ibufs.at[jnp.int32(b), jnp.int32(1)]],
                        add=True)

                    @pl.when(jb + 3 < nb)
                    def _pref():
                        _ifetch(jb + 3, (b + 3) & 3).start()

                    @pl.when(jb + 2 < nb)
                    def _next():
                        _ifetch(jb + 2, (b + 2) & 3).wait()
                        _gath(jb + 2, (b + 2) & 3, b & 1).start()

                if b == 0:
                    _one()
                else:
                    pl.when(jb < nb)(_one)
        plsc.subcore_barrier()

        pltpu.sync_copy(acc_sh.at[pl.ds(base, ROWS_PER_SUB)],
                        out_hbm.at[cid, pl.ds(base, ROWS_PER_SUB)])

    return body(src_emb, eidx)


def _merge_partials(partials):
    blk = 400

    def body(p_ref, o_ref):
        o_ref[...] = jnp.sum(p_ref[...], axis=0)

    return pl.pallas_call(
        body,
        out_shape=jax.ShapeDtypeStruct((N_DST, D), jnp.float32),
        grid=(N_DST // blk,),
        in_specs=[pl.BlockSpec((NC, blk, D), lambda i: (i * 0, i, i * 0))],
        out_specs=pl.BlockSpec((blk, D), lambda i: (i, i * 0)),
    )(partials)


def kernel(src_emb, edge_index):
    n_src = src_emb.shape[0]
    e = edge_index.shape[1]
    nb = -(-e // (NW * BATCH))
    e_pad = NW * BATCH * nb
    npad = e_pad - e

    ei = edge_index.astype(jnp.int32)
    pad = jnp.arange(npad, dtype=jnp.int32)
    sidx = jnp.concatenate([ei[0], pad % n_src]).reshape(NW, nb, BATCH)
    didx = jnp.concatenate([ei[1], N_DST + (pad % EXTRA)]).reshape(NW, nb, BATCH)
    eidx = jnp.stack([sidx, didx], axis=2)

    partials = _sc_partial_sums(src_emb.astype(jnp.float32), eidx, nb)
    return _merge_partials(partials)


# speedup vs baseline: 13.3408x; 1.0234x over previous
import functools

import jax
import jax.numpy as jnp
from jax import lax
from jax.experimental import pallas as pl
from jax.experimental.pallas import tpu as pltpu
from jax.experimental.pallas import tpu_sc as plsc

_INFO = plsc.get_sparse_core_info()
NC = _INFO.num_cores
NS = _INFO.num_subcores
L = _INFO.num_lanes
NW = NC * NS

N_DST = 10000
D = 128
BATCH = 128
ACC_ROWS = 10112
ROWS_PER_SUB = ACC_ROWS // NS


def _i32(x):
    return jnp.int32(x)


def _sc_partial_sums(src_emb, sidx, didx, ept, nbf, tail):
    mesh = plsc.VectorSubcoreMesh(core_axis_name="c", subcore_axis_name="s")

    assert nbf >= 3 and 0 < tail <= L and tail % 8 == 0

    @functools.partial(
        pl.kernel,
        mesh=mesh,
        out_type=jax.ShapeDtypeStruct((NC, ACC_ROWS, D), jnp.float32),
        scratch_types=[
            pltpu.VMEM((4, 2, BATCH), jnp.int32),
            pltpu.VMEM((2, L), jnp.int32),
            pltpu.VMEM((2, BATCH, D), jnp.float32),
            pltpu.VMEM_SHARED((ACC_ROWS, D), jnp.float32),
            pltpu.SemaphoreType.DMA,
            pltpu.SemaphoreType.DMA,
            pltpu.SemaphoreType.DMA,
            pltpu.SemaphoreType.DMA,
            pltpu.SemaphoreType.DMA,
            pltpu.SemaphoreType.DMA,
        ],
    )
    def body(src_hbm, sidx_hbm, didx_hbm, out_hbm, ibufs, tidx, rows_v,
             acc_sh, is0, is1, is2, is3, gs0, gs1):
        cid = lax.axis_index("c")
        sid = lax.axis_index("s")
        wid = sid * NC + cid
        tile_base = wid * ept
        isems = (is0, is1, is2, is3)
        gsems = (gs0, gs1)

        def _ifetch(jb, k):
            off = tile_base + jb * BATCH
            return (
                pltpu.make_async_copy(
                    sidx_hbm.at[pl.ds(off, BATCH)],
                    ibufs.at[_i32(k), _i32(0)], isems[k]),
                pltpu.make_async_copy(
                    didx_hbm.at[pl.ds(off, BATCH)],
                    ibufs.at[_i32(k), _i32(1)], isems[k]),
            )

        def _istart(jb, k):
            a, b = _ifetch(jb, k)
            a.start()
            b.start()

        def _iwait(jb, k):
            a, b = _ifetch(jb, k)
            a.wait()
            b.wait()

        def _gath(jb, k, b):
            return pltpu.make_async_copy(
                src_hbm.at[ibufs.at[_i32(k), _i32(0)]],
                rows_v.at[_i32(b)], gsems[b])

        @pl.loop(_i32(0), _i32(BATCH))
        def _zrow(i):
            for c in range(D // L):
                rows_v[_i32(0), i, pl.ds(c * L, L)] = jnp.zeros(
                    (L,), jnp.float32)

        base = sid * ROWS_PER_SUB
        full = ROWS_PER_SUB // BATCH
        rem = ROWS_PER_SUB - full * BATCH
        for k in range(full):
            pltpu.sync_copy(rows_v.at[_i32(0)],
                            acc_sh.at[pl.ds(base + k * BATCH, BATCH)])
        if rem:
            pltpu.sync_copy(rows_v.at[_i32(0)].at[pl.ds(0, rem)],
                            acc_sh.at[pl.ds(base + full * BATCH, rem)])
        plsc.subcore_barrier()

        _istart(_i32(0), 0)
        _istart(_i32(1), 1)
        _istart(_i32(2), 2)
        _iwait(_i32(0), 0)
        _gath(_i32(0), 0, 0).start()
        _iwait(_i32(1), 1)
        _gath(_i32(1), 1, 1).start()

        @pl.loop(_i32(0), _i32(nbf), step=_i32(4))
        def _step(j):
            for b in range(4):
                jb = j + b

                def _one(jb=jb, b=b):
                    _gath(jb, b, b & 1).wait()
                    pltpu.sync_copy(
                        rows_v.at[_i32(b & 1)],
                        acc_sh.at[ibufs.at[_i32(b), _i32(1)]],
                        add=True)

                    @pl.when(jb + 3 < nbf)
                    def _pref():
                        _istart(jb + 3, (b + 3) & 3)

                    @pl.when(jb + 2 < nbf)
                    def _next():
                        _iwait(jb + 2, (b + 2) & 3)
                        _gath(jb + 2, (b + 2) & 3, b & 1).start()

                if b == 0:
                    _one()
                else:
                    pl.when(jb < nbf)(_one)

        toff = tile_base + nbf * BATCH
        pltpu.sync_copy(sidx_hbm.at[pl.ds(toff, tail)], tidx.at[_i32(0)])
        pltpu.sync_copy(didx_hbm.at[pl.ds(toff, tail)], tidx.at[_i32(1)])
        pltpu.async_copy(src_hbm.at[tidx.at[_i32(0)]],
                         rows_v.at[_i32(0)].at[pl.ds(0, tail)], gs0).wait()
        pltpu.sync_copy(rows_v.at[_i32(0)].at[pl.ds(0, tail)],
                        acc_sh.at[tidx.at[_i32(1)]], add=True)
        plsc.subcore_barrier()

        pltpu.sync_copy(acc_sh.at[pl.ds(base, ROWS_PER_SUB)],
                        out_hbm.at[cid, pl.ds(base, ROWS_PER_SUB)])

    return body(src_emb, sidx, didx)


def _merge_partials(partials):
    blk = 400

    def body(p_ref, o_ref):
        o_ref[...] = jnp.sum(p_ref[...], axis=0)

    return pl.pallas_call(
        body,
        out_shape=jax.ShapeDtypeStruct((N_DST, D), jnp.float32),
        grid=(N_DST // blk,),
        in_specs=[pl.BlockSpec((NC, blk, D), lambda i: (i * 0, i, i * 0))],
        out_specs=pl.BlockSpec((blk, D), lambda i: (i, i * 0)),
    )(partials)


def kernel(src_emb, edge_index):
    e = edge_index.shape[1]
    assert e % NW == 0
    ept = e // NW
    nbf = ept // BATCH
    tail = ept - nbf * BATCH

    sidx = edge_index[0].astype(jnp.int32)
    didx = edge_index[1].astype(jnp.int32)
    partials = _sc_partial_sums(src_emb, sidx, didx, ept, nbf, tail)
    return _merge_partials(partials)


# speedup vs baseline: 14.0266x; 1.0514x over previous
import functools

import jax
import jax.numpy as jnp
from jax import lax
from jax.experimental import pallas as pl
from jax.experimental.pallas import tpu as pltpu
from jax.experimental.pallas import tpu_sc as plsc

_INFO = plsc.get_sparse_core_info()
NC = _INFO.num_cores
NS = _INFO.num_subcores
L = _INFO.num_lanes
NW = NC * NS

N_DST = 10000
D = 128
BATCH = 128
SHARE0 = 640
SHARE = 624
assert SHARE0 + (NS - 1) * SHARE == N_DST


def _i32(x):
    return jnp.int32(x)


def _sc_partial_sums(src_emb, sidx, didx, ept, nbf, tail):
    mesh = plsc.VectorSubcoreMesh(core_axis_name="c", subcore_axis_name="s")

    assert nbf >= 3 and 0 < tail <= L and tail % 8 == 0

    @functools.partial(
        pl.kernel,
        mesh=mesh,
        out_type=jax.ShapeDtypeStruct((NC, N_DST, D), jnp.float32),
        scratch_types=[
            pltpu.VMEM((4, 2, BATCH), jnp.int32),
            pltpu.VMEM((3, BATCH, D), jnp.float32),
            pltpu.VMEM_SHARED((N_DST, D), jnp.float32),
            pltpu.SemaphoreType.DMA,
            pltpu.SemaphoreType.DMA,
            pltpu.SemaphoreType.DMA,
            pltpu.SemaphoreType.DMA,
            pltpu.SemaphoreType.DMA,
            pltpu.SemaphoreType.DMA,
            pltpu.SemaphoreType.DMA,
            pltpu.SemaphoreType.DMA,
            pltpu.SemaphoreType.DMA,
            pltpu.SemaphoreType.DMA,
        ],
    )
    def body(src_hbm, sidx_hbm, didx_hbm, out_hbm, ibufs, rows_v, acc_sh,
             is0, is1, is2, is3, gs0, gs1, gs2, ss0, ss1, ss2):
        cid = lax.axis_index("c")
        sid = lax.axis_index("s")
        wid = sid * NC + cid
        tile_base = wid * ept
        isems = (is0, is1, is2, is3)
        gsems = (gs0, gs1, gs2)
        ssems = (ss0, ss1, ss2)

        def _ifetch(jb, k):
            off = tile_base + jb * BATCH
            return (
                pltpu.make_async_copy(
                    sidx_hbm.at[pl.ds(off, BATCH)],
                    ibufs.at[_i32(k), _i32(0)], isems[k]),
                pltpu.make_async_copy(
                    didx_hbm.at[pl.ds(off, BATCH)],
                    ibufs.at[_i32(k), _i32(1)], isems[k]),
            )

        def _istart(jb, k):
            a, b = _ifetch(jb, k)
            a.start()
            b.start()

        def _iwait(jb, k):
            a, b = _ifetch(jb, k)
            a.wait()
            b.wait()

        def _gath(k4, r3):
            return pltpu.make_async_copy(
                src_hbm.at[ibufs.at[_i32(k4), _i32(0)]],
                rows_v.at[_i32(r3)], gsems[r3])

        def _scat_start(k4, r3):
            pltpu.async_copy(
                rows_v.at[_i32(r3)],
                acc_sh.at[ibufs.at[_i32(k4), _i32(1)]],
                ssems[r3], add=True)

        def _scat_wait(k4, r3):
            pltpu.make_async_copy(
                rows_v.at[_i32(r3)],
                acc_sh.at[ibufs.at[_i32(k4), _i32(1)]],
                ssems[r3]).wait()

        @pl.loop(_i32(0), _i32(BATCH))
        def _zrow(i):
            for c in range(D // L):
                rows_v[_i32(0), i, pl.ds(c * L, L)] = jnp.zeros(
                    (L,), jnp.float32)

        @pl.when(sid == 0)
        def _zero0():
            for k in range(SHARE0 // BATCH):
                pltpu.sync_copy(rows_v.at[_i32(0)],
                                acc_sh.at[pl.ds(k * BATCH, BATCH)])

        @pl.when(sid > 0)
        def _zero():
            zbase = SHARE0 + (sid - 1) * SHARE
            for k in range(SHARE // BATCH):
                pltpu.sync_copy(rows_v.at[_i32(0)],
                                acc_sh.at[pl.ds(zbase + k * BATCH, BATCH)])
            zrem = SHARE - (SHARE // BATCH) * BATCH
            if zrem:
                pltpu.sync_copy(
                    rows_v.at[_i32(0)].at[pl.ds(0, zrem)],
                    acc_sh.at[pl.ds(zbase + (SHARE // BATCH) * BATCH, zrem)])

        plsc.subcore_barrier()

        _istart(_i32(0), 0)
        _istart(_i32(1), 1)
        _istart(_i32(2), 2)
        _iwait(_i32(0), 0)
        _gath(0, 0).start()
        _iwait(_i32(1), 1)
        _gath(1, 1).start()

        @pl.loop(_i32(0), _i32(nbf), step=_i32(12))
        def _step(j):
            for b in range(12):
                jb = j + b

                def _one(jb=jb, b=b):
                    _gath(b % 4, b % 3).wait()
                    _scat_start(b % 4, b % 3)

                    if b == 0:
                        @pl.when(jb > 0)
                        def _drain():
                            _scat_wait((b - 1) % 4, (b - 1) % 3)
                    else:
                        _scat_wait((b - 1) % 4, (b - 1) % 3)

                    @pl.when(jb + 3 < nbf)
                    def _pref():
                        _istart(jb + 3, (b + 3) % 4)

                    @pl.when(jb + 2 < nbf)
                    def _next():
                        _iwait(jb + 2, (b + 2) % 4)
                        _gath((b + 2) % 4, (b + 2) % 3).start()

                if b == 0:
                    _one()
                else:
                    pl.when(jb < nbf)(_one)

        _scat_wait((nbf - 1) % 4, (nbf - 1) % 3)
        toff = tile_base + nbf * BATCH
        pltpu.sync_copy(sidx_hbm.at[pl.ds(toff, tail)],
                        ibufs.at[_i32(0), _i32(0)].at[pl.ds(0, tail)])
        pltpu.sync_copy(didx_hbm.at[pl.ds(toff, tail)],
                        ibufs.at[_i32(0), _i32(1)].at[pl.ds(0, tail)])
        vs = ibufs[_i32(0), _i32(0), pl.ds(0, L)]
        vd = ibufs[_i32(0), _i32(1), pl.ds(0, L)]
        pltpu.async_copy(src_hbm.at[vs],
                         rows_v.at[_i32(0)].at[pl.ds(0, tail)], gs0).wait()
        pltpu.sync_copy(rows_v.at[_i32(0)].at[pl.ds(0, tail)],
                        acc_sh.at[vd], add=True)
        plsc.subcore_barrier()

        @pl.when(sid == 0)
        def _pub0():
            pltpu.sync_copy(acc_sh.at[pl.ds(0, SHARE0)],
                            out_hbm.at[cid, pl.ds(0, SHARE0)])

        @pl.when(sid > 0)
        def _pub():
            pbase = SHARE0 + (sid - 1) * SHARE
            pltpu.sync_copy(acc_sh.at[pl.ds(pbase, SHARE)],
                            out_hbm.at[cid, pl.ds(pbase, SHARE)])

    return body(src_emb, sidx, didx)


def _merge_partials(partials):
    blk = 400

    def body(p_ref, o_ref):
        o_ref[...] = jnp.sum(p_ref[...], axis=0)

    return pl.pallas_call(
        body,
        out_shape=jax.ShapeDtypeStruct((N_DST, D), jnp.float32),
        grid=(N_DST // blk,),
        in_specs=[pl.BlockSpec((NC, blk, D), lambda i: (i * 0, i, i * 0))],
        out_specs=pl.BlockSpec((blk, D), lambda i: (i, i * 0)),
    )(partials)


def kernel(src_emb, edge_index):
    e = edge_index.shape[1]
    assert e % NW == 0
    ept = e // NW
    nbf = ept // BATCH
    tail = ept - nbf * BATCH

    sidx = edge_index[0].astype(jnp.int32)
    didx = edge_index[1].astype(jnp.int32)
    partials = _sc_partial_sums(src_emb, sidx, didx, ept, nbf, tail)
    return _merge_partials(partials)
